# baseline (speedup 1.0000x reference)
import functools

import jax
import jax.numpy as jnp
from jax.experimental import pallas as pl
from jax.experimental.pallas import tpu as pltpu

_PACK = 64
_BLOCK_ROWS = 1024


def _linear_block_kernel(x_ref, w_ref, b_ref, o_ref):
    acc = jnp.dot(
        x_ref[...].astype(jnp.bfloat16),
        w_ref[...],
        preferred_element_type=jnp.float32,
    )
    o_ref[...] = (acc + b_ref[...]).astype(o_ref.dtype)


@functools.partial(jax.jit, static_argnames=("block_rows",))
def _packed_linear(x2d, w_exp, b_exp, *, block_rows):
    rows, c_in = x2d.shape
    c_out = w_exp.shape[1]
    grid = (pl.cdiv(rows, block_rows),)
    return pl.pallas_call(
        _linear_block_kernel,
        out_shape=jax.ShapeDtypeStruct((rows, c_out), x2d.dtype),
        grid=grid,
        in_specs=[
            pl.BlockSpec((block_rows, c_in), lambda i: (i, 0)),
            pl.BlockSpec((w_exp.shape[0], c_out), lambda i: (0, 0)),
            pl.BlockSpec((1, c_out), lambda i: (0, 0)),
        ],
        out_specs=pl.BlockSpec((block_rows, c_out), lambda i: (i, 0)),
        compiler_params=pltpu.CompilerParams(
            dimension_semantics=("parallel",)),
    )(x2d, w_exp, b_exp)


def kernel(x, weight, bias):
    B, in_f = x.shape
    out_f = weight.shape[0]
    assert B % _PACK == 0, "batch must fold exactly into 64-row packs"

    eye = jnp.eye(_PACK, dtype=jnp.bfloat16)
    w_exp = (eye[:, None, :, None] * weight.T.astype(jnp.bfloat16)[None, :, None, :]
             ).reshape(_PACK * in_f, _PACK * out_f)
    b_exp = jnp.tile(bias, _PACK).reshape(1, _PACK * out_f)

    x2d = x.reshape(B // _PACK, _PACK * in_f)
    y2d = _packed_linear(x2d, w_exp, b_exp, block_rows=_BLOCK_ROWS)
    return y2d.reshape(B, out_f)


# speedup vs baseline: 4.9983x; 4.9983x over previous
import functools

import jax
import jax.numpy as jnp
from jax import lax
from jax.experimental import pallas as pl
from jax.experimental.pallas import tpu as pltpu

_BLOCK_ROWS = 16384


def _linear_kernel(x_ref, w_ref, b_ref, o_ref):
    acc = lax.dot_general(
        x_ref[...],
        w_ref[...],
        (((1,), (1,)), ((), ())),
        preferred_element_type=jnp.float32,
    )
    o_ref[...] = (acc + b_ref[...]).astype(o_ref.dtype)


@functools.partial(jax.jit, static_argnames=("block_rows",))
def _native_linear(x, weight, bias2d, *, block_rows):
    rows, in_f = x.shape
    out_f = weight.shape[0]
    grid = (pl.cdiv(rows, block_rows),)
    return pl.pallas_call(
        _linear_kernel,
        out_shape=jax.ShapeDtypeStruct((rows, out_f), x.dtype),
        grid=grid,
        in_specs=[
            pl.BlockSpec((block_rows, in_f), lambda i: (i, 0)),
            pl.BlockSpec((weight.shape[0], in_f), lambda i: (0, 0)),
            pl.BlockSpec((1, out_f), lambda i: (0, 0)),
        ],
        out_specs=pl.BlockSpec((block_rows, out_f), lambda i: (i, 0)),
        compiler_params=pltpu.CompilerParams(
            dimension_semantics=("parallel",)),
    )(x, weight, bias2d)


def kernel(x, weight, bias):
    return _native_linear(x, weight, bias.reshape(1, -1),
                          block_rows=_BLOCK_ROWS)
